# Initial kernel scaffold; baseline (speedup 1.0000x reference)
import functools

import jax
import jax.numpy as jnp
from jax import lax
from jax.experimental import pallas as pl
from jax.experimental.pallas import tpu as pltpu
from jax.experimental.pallas import tpu_sc as plsc

N = 10000
E = 320000
D = 128

NW = 32
NSUB = 16
EPW = E // NW
CH = 128
NCHUNK = -(-EPW // CH)
EPW_PAD = NCHUNK * CH
N_PAD = 10112
RPT = N_PAD // NSUB

_mesh = plsc.VectorSubcoreMesh(core_axis_name="c", subcore_axis_name="s")


@functools.partial(
    pl.kernel,
    mesh=_mesh,
    out_type=jax.ShapeDtypeStruct((2, 2, N_PAD), jnp.float32),
    scratch_types=[
        pltpu.VMEM((NCHUNK, CH), jnp.int32),
        pltpu.VMEM((CH,), jnp.float32),
        pltpu.VMEM_SHARED((N_PAD,), jnp.float32),
        pltpu.VMEM_SHARED((N_PAD,), jnp.float32),
    ],
)
def _deg_kernel(src_hbm, dst_hbm, ones_hbm, zrow_hbm, out_hbm,
                idx_v, ones_v, dego, degi):
    c = lax.axis_index("c")
    s = lax.axis_index("s")
    wid = c * NSUB + s
    base = s * RPT
    pltpu.sync_copy(ones_hbm, ones_v)
    pltpu.sync_copy(zrow_hbm, dego.at[pl.ds(base, RPT)])
    pltpu.sync_copy(zrow_hbm, degi.at[pl.ds(base, RPT)])
    plsc.subcore_barrier()

    pltpu.sync_copy(src_hbm.at[wid], idx_v)

    @pl.loop(0, NCHUNK)
    def _(j):
        pltpu.sync_copy(ones_v, dego.at[idx_v.at[j]], add=True)

    pltpu.sync_copy(dst_hbm.at[wid], idx_v)

    @pl.loop(0, NCHUNK)
    def _(j):
        pltpu.sync_copy(ones_v, degi.at[idx_v.at[j]], add=True)

    plsc.subcore_barrier()
    pltpu.sync_copy(dego.at[pl.ds(base, RPT)], out_hbm.at[c, 0, pl.ds(base, RPT)])
    pltpu.sync_copy(degi.at[pl.ds(base, RPT)], out_hbm.at[c, 1, pl.ds(base, RPT)])


@functools.partial(
    pl.kernel,
    mesh=_mesh,
    out_type=jax.ShapeDtypeStruct((2, N_PAD, D), jnp.float32),
    scratch_types=[
        pltpu.VMEM((NCHUNK, CH), jnp.int32),
        pltpu.VMEM((NCHUNK, CH), jnp.int32),
        pltpu.VMEM((CH, D), jnp.float32),
        pltpu.VMEM_SHARED((N_PAD, D), jnp.float32),
    ],
)
def _agg_kernel(h_hbm, src_hbm, dst_hbm, zrows_hbm, out_hbm,
                sidx, didx, rows, acc):
    c = lax.axis_index("c")
    s = lax.axis_index("s")
    wid = c * NSUB + s
    base = s * RPT
    pltpu.sync_copy(zrows_hbm, acc.at[pl.ds(base, RPT)])
    pltpu.sync_copy(src_hbm.at[wid], sidx)
    pltpu.sync_copy(dst_hbm.at[wid], didx)
    plsc.subcore_barrier()

    @pl.loop(0, NCHUNK)
    def _(j):
        pltpu.sync_copy(h_hbm.at[sidx.at[j]], rows)
        pltpu.sync_copy(rows, acc.at[didx.at[j]], add=True)

    plsc.subcore_barrier()
    pltpu.sync_copy(acc.at[pl.ds(base, RPT)], out_hbm.at[c, pl.ds(base, RPT)])


def _norm_body(degp_ref, out_ref):
    d = degp_ref[0] + degp_ref[1]
    out_ref[...] = lax.rsqrt(jnp.maximum(d, 1.0))


_norm = pl.pallas_call(
    _norm_body,
    out_shape=jax.ShapeDtypeStruct((2, N_PAD), jnp.float32),
)


def _mm1_body(x_ref, w_ref, no_ref, o_ref):
    h = jnp.dot(x_ref[...], w_ref[...], preferred_element_type=jnp.float32)
    o_ref[...] = h * no_ref[...]


_mm1 = pl.pallas_call(
    _mm1_body,
    out_shape=jax.ShapeDtypeStruct((N_PAD, D), jnp.float32),
)


def _mm2_body(p_ref, ni_ref, b_ref, w_ref, no_ref, o_ref):
    x = (p_ref[0] + p_ref[1]) * ni_ref[...] + b_ref[...]
    h = jnp.dot(x, w_ref[...], preferred_element_type=jnp.float32)
    o_ref[...] = h * no_ref[...]


_mm2 = pl.pallas_call(
    _mm2_body,
    out_shape=jax.ShapeDtypeStruct((N_PAD, D), jnp.float32),
)


def _fin_body(p_ref, ni_ref, b_ref, o_ref):
    o_ref[...] = (p_ref[0] + p_ref[1]) * ni_ref[...] + b_ref[...]


_fin = pl.pallas_call(
    _fin_body,
    out_shape=jax.ShapeDtypeStruct((N_PAD, D), jnp.float32),
)


def kernel(feat, edge_index, W1, b1, W2, b2):
    src = edge_index[0]
    dst = edge_index[1]
    pad = EPW_PAD - EPW
    s3 = jnp.pad(src.reshape(NW, EPW), ((0, 0), (0, pad)),
                 constant_values=N).reshape(NW, NCHUNK, CH)
    d3 = jnp.pad(dst.reshape(NW, EPW), ((0, 0), (0, pad)),
                 constant_values=N).reshape(NW, NCHUNK, CH)
    ones_row = jnp.ones((CH,), jnp.float32)
    zrow = jnp.zeros((RPT,), jnp.float32)
    zrows = jnp.zeros((RPT, D), jnp.float32)
    featp = jnp.pad(feat, ((0, N_PAD - N), (0, 0)))

    degp = _deg_kernel(s3, d3, ones_row, zrow)
    norms = _norm(degp)
    no = norms[0].reshape(N_PAD, 1)
    ni = norms[1].reshape(N_PAD, 1)

    h1 = _mm1(featp, W1, no)
    p1 = _agg_kernel(h1, s3, d3, zrows)
    h2 = _mm2(p1, ni, b1.reshape(1, D), W2, no)
    p2 = _agg_kernel(h2, s3, d3, zrows)
    outp = _fin(p2, ni, b2.reshape(1, D))
    return outp[:N]


# speedup vs baseline: 5.1820x; 5.1820x over previous
import functools

import jax
import jax.numpy as jnp
from jax import lax
from jax.experimental import pallas as pl
from jax.experimental.pallas import tpu as pltpu
from jax.experimental.pallas import tpu_sc as plsc

N = 10000
E = 320000
D = 128

NW = 32
NSUB = 16
EPW = E // NW
CH = 128
NCHUNK = -(-EPW // CH)
EPW_PAD = NCHUNK * CH
N_PAD = 10112
RPT = N_PAD // NSUB

_mesh = plsc.VectorSubcoreMesh(core_axis_name="c", subcore_axis_name="s")


@functools.partial(
    pl.kernel,
    mesh=_mesh,
    out_type=tuple(jax.ShapeDtypeStruct((N_PAD,), jnp.float32)
                   for _ in range(4)),
    scratch_types=[
        pltpu.VMEM((NCHUNK, CH), jnp.int32),
        pltpu.VMEM((CH,), jnp.float32),
        pltpu.VMEM((RPT,), jnp.float32),
        pltpu.VMEM_SHARED((N_PAD,), jnp.float32),
        pltpu.VMEM_SHARED((N_PAD,), jnp.float32),
    ],
)
def _deg_kernel(src_hbm, dst_hbm, ones_hbm, zrow_hbm,
                dego0_hbm, degi0_hbm, dego1_hbm, degi1_hbm,
                idx_v, ones_v, vbuf, dego, degi):
    c = lax.axis_index("c")
    s = lax.axis_index("s")
    wid = c * NSUB + s
    base = s * RPT
    pltpu.sync_copy(ones_hbm, ones_v)
    pltpu.sync_copy(zrow_hbm, vbuf)
    pltpu.sync_copy(vbuf, dego.at[pl.ds(base, RPT)])
    pltpu.sync_copy(vbuf, degi.at[pl.ds(base, RPT)])
    plsc.subcore_barrier()

    pltpu.sync_copy(src_hbm.at[wid], idx_v)

    @pl.loop(0, NCHUNK)
    def _(j):
        pltpu.sync_copy(ones_v, dego.at[idx_v.at[j]], add=True)

    pltpu.sync_copy(dst_hbm.at[wid], idx_v)

    @pl.loop(0, NCHUNK)
    def _(j):
        pltpu.sync_copy(ones_v, degi.at[idx_v.at[j]], add=True)

    plsc.subcore_barrier()

    @pl.when(c == 0)
    def _():
        pltpu.sync_copy(dego.at[pl.ds(base, RPT)], vbuf)
        pltpu.sync_copy(vbuf, dego0_hbm.at[pl.ds(base, RPT)])
        pltpu.sync_copy(degi.at[pl.ds(base, RPT)], vbuf)
        pltpu.sync_copy(vbuf, degi0_hbm.at[pl.ds(base, RPT)])

    @pl.when(c == 1)
    def _():
        pltpu.sync_copy(dego.at[pl.ds(base, RPT)], vbuf)
        pltpu.sync_copy(vbuf, dego1_hbm.at[pl.ds(base, RPT)])
        pltpu.sync_copy(degi.at[pl.ds(base, RPT)], vbuf)
        pltpu.sync_copy(vbuf, degi1_hbm.at[pl.ds(base, RPT)])


@functools.partial(
    pl.kernel,
    mesh=_mesh,
    out_type=jax.ShapeDtypeStruct((2, N_PAD, D), jnp.float32),
    scratch_types=[
        pltpu.VMEM((NCHUNK, CH), jnp.int32),
        pltpu.VMEM((NCHUNK, CH), jnp.int32),
        pltpu.VMEM((CH, D), jnp.float32),
        pltpu.VMEM_SHARED((N_PAD, D), jnp.float32),
    ],
)
def _agg_kernel(h_hbm, src_hbm, dst_hbm, zrows_hbm, out_hbm,
                sidx, didx, rows, acc):
    c = lax.axis_index("c")
    s = lax.axis_index("s")
    wid = c * NSUB + s
    base = s * RPT
    pltpu.sync_copy(zrows_hbm, rows)
    for k in range(-(-RPT // CH)):
        sz = min(CH, RPT - k * CH)
        pltpu.sync_copy(rows.at[pl.ds(0, sz)], acc.at[pl.ds(base + k * CH, sz)])
    pltpu.sync_copy(src_hbm.at[wid], sidx)
    pltpu.sync_copy(dst_hbm.at[wid], didx)
    plsc.subcore_barrier()

    @pl.loop(0, NCHUNK)
    def _(j):
        pltpu.sync_copy(h_hbm.at[sidx.at[j]], rows)
        pltpu.sync_copy(rows, acc.at[didx.at[j]], add=True)

    plsc.subcore_barrier()
    for k in range(-(-RPT // CH)):
        sz = min(CH, RPT - k * CH)
        pltpu.sync_copy(acc.at[pl.ds(base + k * CH, sz)], rows.at[pl.ds(0, sz)])
        pltpu.sync_copy(rows.at[pl.ds(0, sz)],
                        out_hbm.at[c, pl.ds(base + k * CH, sz)])


def _norm_body(do0, di0, do1, di1, no_ref, ni_ref):
    no_ref[...] = lax.rsqrt(jnp.maximum(do0[...] + do1[...], 1.0))
    ni_ref[...] = lax.rsqrt(jnp.maximum(di0[...] + di1[...], 1.0))


_norm = pl.pallas_call(
    _norm_body,
    out_shape=(jax.ShapeDtypeStruct((N_PAD,), jnp.float32),
               jax.ShapeDtypeStruct((N_PAD,), jnp.float32)),
)


def _mm1_body(x_ref, w_ref, no_ref, o_ref):
    h = jnp.dot(x_ref[...], w_ref[...], preferred_element_type=jnp.float32)
    o_ref[...] = h * no_ref[...]


_mm1 = pl.pallas_call(
    _mm1_body,
    out_shape=jax.ShapeDtypeStruct((N_PAD, D), jnp.float32),
)


def _mm2_body(p_ref, ni_ref, b_ref, w_ref, no_ref, o_ref):
    x = (p_ref[0] + p_ref[1]) * ni_ref[...] + b_ref[...]
    h = jnp.dot(x, w_ref[...], preferred_element_type=jnp.float32)
    o_ref[...] = h * no_ref[...]


_mm2 = pl.pallas_call(
    _mm2_body,
    out_shape=jax.ShapeDtypeStruct((N_PAD, D), jnp.float32),
)


def _fin_body(p_ref, ni_ref, b_ref, o_ref):
    o_ref[...] = (p_ref[0] + p_ref[1]) * ni_ref[...] + b_ref[...]


_fin = pl.pallas_call(
    _fin_body,
    out_shape=jax.ShapeDtypeStruct((N_PAD, D), jnp.float32),
)


def kernel(feat, edge_index, W1, b1, W2, b2):
    src = edge_index[0]
    dst = edge_index[1]
    pad = EPW_PAD - EPW
    s3 = jnp.pad(src.reshape(NW, EPW), ((0, 0), (0, pad)),
                 constant_values=N).reshape(NW, NCHUNK, CH)
    d3 = jnp.pad(dst.reshape(NW, EPW), ((0, 0), (0, pad)),
                 constant_values=N).reshape(NW, NCHUNK, CH)
    ones_row = jnp.ones((CH,), jnp.float32)
    zrow = jnp.zeros((RPT,), jnp.float32)
    zrows = jnp.zeros((CH, D), jnp.float32)
    featp = jnp.pad(feat, ((0, N_PAD - N), (0, 0)))

    do0, di0, do1, di1 = _deg_kernel(s3, d3, ones_row, zrow)
    no, ni = _norm(do0, di0, do1, di1)
    no = no.reshape(N_PAD, 1)
    ni = ni.reshape(N_PAD, 1)

    h1 = _mm1(featp, W1, no)
    p1 = _agg_kernel(h1, s3, d3, zrows)
    h2 = _mm2(p1, ni, b1.reshape(1, D), W2, no)
    p2 = _agg_kernel(h2, s3, d3, zrows)
    outp = _fin(p2, ni, b2.reshape(1, D))
    return outp[:N]
